# baseline (speedup 1.0000x reference)
import functools

import jax
import jax.numpy as jnp
from jax import lax
from jax.experimental import pallas as pl
from jax.experimental.pallas import tpu as pltpu
from jax.experimental.pallas import tpu_sc as plsc

_LANE = 128


def kernel(x, embedding):
    B, S = x.shape
    V, D = embedding.shape
    N = B * S
    NC, NS = 2, 16
    NW = NC * NS
    rows = 2 * N // _LANE
    rows_w = rows // NW
    R = 16
    n_g = rows_w // R

    mesh = plsc.VectorSubcoreMesh(core_axis_name="c", subcore_axis_name="s")

    @functools.partial(
        pl.kernel,
        mesh=mesh,
        out_type=jax.ShapeDtypeStruct((rows, _LANE), jnp.float32),
        scratch_types=[
            pltpu.VMEM((R, _LANE), jnp.int32),
            pltpu.VMEM((R, _LANE), jnp.float32),
            pltpu.SemaphoreType.DMA,
        ],
    )
    def gather_k(widx_hbm, tab_hbm, out_hbm, idx_v, val_v, sem):
        wid = lax.axis_index("s") * NC + lax.axis_index("c")
        base = wid * rows_w

        def group(g, carry):
            off = base + g * R
            pltpu.sync_copy(widx_hbm.at[pl.ds(off, R)], idx_v)
            cps = [pltpu.async_copy(tab_hbm.at[idx_v.at[j]], val_v.at[j], sem)
                   for j in range(R)]
            for c in cps:
                c.wait()
            pltpu.sync_copy(val_v, out_hbm.at[pl.ds(off, R)])
            return carry

        lax.fori_loop(0, n_g, group, 0)

    widx = (x.reshape(N, 1) * 2 + jnp.arange(2, dtype=jnp.int32)).reshape(
        rows, _LANE)
    out = gather_k(widx, embedding.reshape(2 * V))
    return out.reshape(B, S, D)


# speedup vs baseline: 1.0361x; 1.0361x over previous
import functools

import jax
import jax.numpy as jnp
from jax import lax
from jax.experimental import pallas as pl
from jax.experimental.pallas import tpu as pltpu
from jax.experimental.pallas import tpu_sc as plsc

_LANE = 128
_VL = 16


def kernel(x, embedding):
    B, S = x.shape
    V, D = embedding.shape
    N = B * S
    NC, NS = 2, 16
    NW = NC * NS
    rows = N // _LANE
    rows_w = rows // NW
    R = 16
    n_g = rows_w // R

    mesh = plsc.VectorSubcoreMesh(core_axis_name="c", subcore_axis_name="s")

    @functools.partial(
        pl.kernel,
        mesh=mesh,
        out_type=jax.ShapeDtypeStruct((2 * rows, _LANE), jnp.float32),
        scratch_types=[
            pltpu.VMEM((R, _LANE), jnp.int32),
            pltpu.VMEM((2 * R, _LANE), jnp.int32),
            pltpu.VMEM((2 * R, _LANE), jnp.float32),
            pltpu.SemaphoreType.DMA,
        ],
    )
    def gather_k(idx_hbm, tab_hbm, out_hbm, idx_v, widx_v, val_v, sem):
        wid = lax.axis_index("s") * NC + lax.axis_index("c")
        base = wid * rows_w
        lane = lax.iota(jnp.int32, _VL)
        perm_lo = lax.shift_right_logical(lane, 1)
        perm_hi = perm_lo + 8
        adj = lax.bitwise_and(lane, 1)
        _dnums = lax.GatherDimensionNumbers(
            offset_dims=(), collapsed_slice_dims=(0,), start_index_map=(0,))

        def lane_gather(vec, perm):
            return lax.gather(
                vec, perm.reshape(_VL, 1), dimension_numbers=_dnums,
                slice_sizes=(1,),
                mode=lax.GatherScatterMode.PROMISE_IN_BOUNDS)

        def expand_row(j):
            for t in range(8):
                v = idx_v[j, pl.ds(_VL * t, _VL)]
                d = v * 2
                o0 = lane_gather(d, perm_lo) + adj
                o1 = lane_gather(d, perm_hi) + adj
                r = 2 * j + t // 4
                c = (32 * t) % _LANE
                widx_v[r, pl.ds(c, _VL)] = o0
                widx_v[r, pl.ds(c + _VL, _VL)] = o1

        def group(g, carry):
            off = base + g * R
            pltpu.sync_copy(idx_hbm.at[pl.ds(off, R)], idx_v)
            for j in range(R):
                expand_row(j)
            cps = [pltpu.async_copy(tab_hbm.at[widx_v.at[j]], val_v.at[j], sem)
                   for j in range(2 * R)]
            for c in cps:
                c.wait()
            pltpu.sync_copy(val_v, out_hbm.at[pl.ds(2 * off, 2 * R)])
            return carry

        lax.fori_loop(0, n_g, group, 0)

    out = gather_k(x.reshape(rows, _LANE), embedding.reshape(2 * V))
    return out.reshape(B, S, D)
